# baseline (speedup 1.0000x reference)
import jax
import jax.numpy as jnp
from jax.experimental import pallas as pl
from jax.experimental.pallas import tpu as pltpu

B, N, D, R = 4, 2048, 256, 3
TB = 256
TP = 512
KMAX = N // 2


def _mega_body(w_ref, b_ref, nums_ref, adj0_ref, adj1_ref, adj2_ref, adj3_ref,
               nodes_ref, hid_ref, k_ref, idx_ref, ks_ref, acc_ref):
    bi = pl.program_id(0)
    r = pl.program_id(1)

    @pl.when(r < R)
    def _():
        xwr = jnp.dot(nodes_ref[0], w_ref[0],
                      preferred_element_type=jnp.float32)
        part = jnp.concatenate(
            [jnp.dot(a_ref[0, 0], xwr, preferred_element_type=jnp.float32)
             for a_ref in (adj0_ref, adj1_ref, adj2_ref, adj3_ref)],
            axis=0)

        @pl.when(r == 0)
        def _():
            acc_ref[...] = part

        @pl.when(r > 0)
        def _():
            acc_ref[...] += part

    @pl.when(r == R)
    def _():
        s = jnp.tanh(acc_ref[...] + b_ref[0])
        num = nums_ref[bi]
        k = jnp.ceil(0.5 * num.astype(jnp.float32)).astype(jnp.int32)
        k_ref[bi] = k
        s_row = s.reshape(1, N)
        irow = jax.lax.broadcasted_iota(jnp.int32, (N, 1), 0)
        ones = jnp.ones((1, N), dtype=jnp.float32)

        rank_chunks = []
        for c in range(N // TB):
            sj = jax.lax.slice(s_row, (0, c * TB), (1, (c + 1) * TB))
            jcol = jax.lax.broadcasted_iota(jnp.int32, (1, TB), 1) + c * TB
            cmp = (s > sj) | ((s == sj) & (irow < jcol))
            rank_chunks.append(jnp.dot(ones, cmp.astype(jnp.float32),
                                       preferred_element_type=jnp.float32))
        rank_row = jnp.concatenate(rank_chunks, axis=1).astype(jnp.int32)

        mask_col = (rank_row < k).astype(jnp.float32).reshape(N, 1)
        hid_ref[0] = nodes_ref[0] * (s * mask_col)

        rank_col = rank_row.reshape(N, 1)
        ivals = jax.lax.broadcasted_iota(jnp.int32, (1, N), 1).astype(jnp.float32)
        for c in range(KMAX // TP):
            p = jax.lax.broadcasted_iota(jnp.int32, (1, TP), 1) + c * TP
            onehot = (rank_col == p).astype(jnp.float32)
            sorted_i = jnp.dot(ivals, onehot, preferred_element_type=jnp.float32)
            sorted_s = jnp.dot(s_row, onehot, preferred_element_type=jnp.float32)
            keep = p < k
            idx_ref[0, 0, c * TP:(c + 1) * TP] = jnp.where(
                keep, sorted_i.astype(jnp.int32), -1).reshape(TP)
            ks_ref[0, 0, c * TP:(c + 1) * TP] = jnp.where(
                keep, sorted_s, 0.0).reshape(TP)
        idx_ref[0, 0, KMAX:] = jnp.full((N - KMAX,), -1, jnp.int32)
        ks_ref[0, 0, KMAX:] = jnp.zeros((N - KMAX,), jnp.float32)


@jax.jit
def kernel(nodes, adjacency, batch_node_nums, W, b):
    hidden, knum, keep_idx, keep_score = pl.pallas_call(
        _mega_body,
        grid=(B, R + 1),
        in_specs=[
            pl.BlockSpec((1, D, 1), lambda bb, r: (jnp.minimum(r, R - 1), 0, 0)),
            pl.BlockSpec(memory_space=pltpu.SMEM),
            pl.BlockSpec(memory_space=pltpu.SMEM),
            pl.BlockSpec((1, 1, N // 4, N),
                         lambda bb, r: (bb, jnp.minimum(r, R - 1), 0, 0)),
            pl.BlockSpec((1, 1, N // 4, N),
                         lambda bb, r: (bb, jnp.minimum(r, R - 1), 1, 0)),
            pl.BlockSpec((1, 1, N // 4, N),
                         lambda bb, r: (bb, jnp.minimum(r, R - 1), 2, 0)),
            pl.BlockSpec((1, 1, N // 4, N),
                         lambda bb, r: (bb, jnp.minimum(r, R - 1), 3, 0)),
            pl.BlockSpec((1, N, D), lambda bb, r: (bb, 0, 0)),
        ],
        out_specs=[
            pl.BlockSpec((1, N, D), lambda bb, r: (bb, 0, 0)),
            pl.BlockSpec(memory_space=pltpu.SMEM, block_shape=(B,),
                         index_map=lambda bb, r: (0,)),
            pl.BlockSpec((1, 1, N), lambda bb, r: (bb, 0, 0)),
            pl.BlockSpec((1, 1, N), lambda bb, r: (bb, 0, 0)),
        ],
        out_shape=[
            jax.ShapeDtypeStruct((B, N, D), jnp.float32),
            jax.ShapeDtypeStruct((B,), jnp.int32),
            jax.ShapeDtypeStruct((B, 1, N), jnp.int32),
            jax.ShapeDtypeStruct((B, 1, N), jnp.float32),
        ],
        scratch_shapes=[pltpu.VMEM((N, 1), jnp.float32)],
    )(W, b, batch_node_nums, adjacency, adjacency, adjacency, adjacency, nodes)

    return (hidden, knum, keep_idx.reshape(B, N), keep_score.reshape(B, N))


# speedup vs baseline: 1.0587x; 1.0587x over previous
import jax
import jax.numpy as jnp
from jax.experimental import pallas as pl
from jax.experimental.pallas import tpu as pltpu

B, N, D, R = 4, 2048, 256, 3
TB = 256
TP = 512
KMAX = N // 2


def _mega_body(w_ref, b_ref, nums_ref, adj0_ref, adj1_ref, adj2_ref, adj3_ref,
               nodes_ref, hid_ref, k_ref, idx_ref, ks_ref, acc_ref):
    bi = pl.program_id(0)
    r = pl.program_id(1)
    xwr = jnp.dot(nodes_ref[0], w_ref[0],
                  preferred_element_type=jnp.float32)
    part = jnp.concatenate(
        [jnp.dot(a_ref[0, 0], xwr, preferred_element_type=jnp.float32)
         for a_ref in (adj0_ref, adj1_ref, adj2_ref, adj3_ref)],
        axis=0)

    @pl.when(r == 0)
    def _():
        acc_ref[...] = part

    @pl.when(r > 0)
    def _():
        acc_ref[...] += part

    @pl.when(r == R - 1)
    def _():
        s = jnp.tanh(acc_ref[...] + b_ref[0])
        num = nums_ref[bi]
        k = jnp.ceil(0.5 * num.astype(jnp.float32)).astype(jnp.int32)
        k_ref[bi] = k
        s_row = s.reshape(1, N)
        irow = jax.lax.broadcasted_iota(jnp.int32, (N, 1), 0)
        ones = jnp.ones((1, N), dtype=jnp.float32)

        rank_chunks = []
        for c in range(N // TB):
            sj = jax.lax.slice(s_row, (0, c * TB), (1, (c + 1) * TB))
            jcol = jax.lax.broadcasted_iota(jnp.int32, (1, TB), 1) + c * TB
            cmp = (s > sj) | ((s == sj) & (irow < jcol))
            rank_chunks.append(jnp.dot(ones, cmp.astype(jnp.float32),
                                       preferred_element_type=jnp.float32))
        rank_row = jnp.concatenate(rank_chunks, axis=1).astype(jnp.int32)

        mask_col = (rank_row < k).astype(jnp.float32).reshape(N, 1)
        hid_ref[0] = nodes_ref[0] * (s * mask_col)

        rank_col = rank_row.reshape(N, 1)
        ivals = jax.lax.broadcasted_iota(jnp.int32, (1, N), 1).astype(jnp.float32)
        for c in range(KMAX // TP):
            p = jax.lax.broadcasted_iota(jnp.int32, (1, TP), 1) + c * TP
            onehot = (rank_col == p).astype(jnp.float32)
            sorted_i = jnp.dot(ivals, onehot, preferred_element_type=jnp.float32)
            sorted_s = jnp.dot(s_row, onehot, preferred_element_type=jnp.float32)
            keep = p < k
            idx_ref[0, 0, c * TP:(c + 1) * TP] = jnp.where(
                keep, sorted_i.astype(jnp.int32), -1).reshape(TP)
            ks_ref[0, 0, c * TP:(c + 1) * TP] = jnp.where(
                keep, sorted_s, 0.0).reshape(TP)
        idx_ref[0, 0, KMAX:] = jnp.full((N - KMAX,), -1, jnp.int32)
        ks_ref[0, 0, KMAX:] = jnp.zeros((N - KMAX,), jnp.float32)


@jax.jit
def kernel(nodes, adjacency, batch_node_nums, W, b):
    hidden, knum, keep_idx, keep_score = pl.pallas_call(
        _mega_body,
        grid=(B, R),
        in_specs=[
            pl.BlockSpec((1, D, 1), lambda bb, r: (r, 0, 0)),
            pl.BlockSpec(memory_space=pltpu.SMEM),
            pl.BlockSpec(memory_space=pltpu.SMEM),
            pl.BlockSpec((1, 1, N // 4, N), lambda bb, r: (bb, r, 0, 0)),
            pl.BlockSpec((1, 1, N // 4, N), lambda bb, r: (bb, r, 1, 0)),
            pl.BlockSpec((1, 1, N // 4, N), lambda bb, r: (bb, r, 2, 0)),
            pl.BlockSpec((1, 1, N // 4, N), lambda bb, r: (bb, r, 3, 0)),
            pl.BlockSpec((1, N, D), lambda bb, r: (bb, 0, 0)),
        ],
        out_specs=[
            pl.BlockSpec((1, N, D), lambda bb, r: (bb, 0, 0)),
            pl.BlockSpec(memory_space=pltpu.SMEM, block_shape=(B,),
                         index_map=lambda bb, r: (0,)),
            pl.BlockSpec((1, 1, N), lambda bb, r: (bb, 0, 0)),
            pl.BlockSpec((1, 1, N), lambda bb, r: (bb, 0, 0)),
        ],
        out_shape=[
            jax.ShapeDtypeStruct((B, N, D), jnp.float32),
            jax.ShapeDtypeStruct((B,), jnp.int32),
            jax.ShapeDtypeStruct((B, 1, N), jnp.int32),
            jax.ShapeDtypeStruct((B, 1, N), jnp.float32),
        ],
        scratch_shapes=[pltpu.VMEM((N, 1), jnp.float32)],
    )(W, b, batch_node_nums, adjacency, adjacency, adjacency, adjacency, nodes)

    return (hidden, knum, keep_idx.reshape(B, N), keep_score.reshape(B, N))
